# baseline (speedup 1.0000x reference)
import jax
import jax.numpy as jnp
from jax.experimental import pallas as pl
from jax.experimental.pallas import tpu as pltpu

_BF = jnp.bfloat16
_NCLS = 10
_DR = 0.2


def _fold_w(w, scale):
    kh, kw = w.shape[2], w.shape[3]
    wt = jnp.transpose(w, (2, 3, 1, 0)).reshape(kh * kw * w.shape[1], w.shape[0])
    return (wt * scale).astype(_BF)


def _fold_w3(w, scale):
    wt = jnp.transpose(w, (2, 3, 1, 0)).reshape(9, w.shape[1], w.shape[0])
    return (wt * scale).astype(_BF)


def _conv1_pool(q, w4, bias, n, ho, wo):
    hp, wp = ho // 2, wo // 2
    cout = w4.shape[2]

    def body(q_ref, w_ref, b_ref, o_ref):
        qv = q_ref[...][0]
        acc = None
        for d in range(4):
            lhs = qv[d:d + ho].reshape(ho * wo, 48)
            z = jnp.dot(lhs, w_ref[d], preferred_element_type=jnp.float32)
            acc = z if acc is None else acc + z
        y = jnp.maximum(acc + b_ref[...], 0.0).astype(_BF).reshape(ho, wo, cout)
        neg = jnp.asarray(-jnp.inf, _BF)
        yp = jnp.pad(y, ((1, 1), (1, 1), (0, 0)), constant_values=neg)
        cmax = None
        for j in range(3):
            cj = yp[:, j:j + 2 * wp].reshape(ho + 2, wp, 2, cout)[:, :, 0]
            cmax = cj if cmax is None else jnp.maximum(cmax, cj)
        out = None
        for i in range(3):
            ri = cmax[i:i + 2 * hp].reshape(hp, 2, wp, cout)[:, 0]
            out = ri if out is None else jnp.maximum(out, ri)
        o_ref[...] = jnp.pad(out, ((1, 1), (1, 1), (0, 0)))[None]

    return pl.pallas_call(
        body,
        grid=(n,),
        in_specs=[
            pl.BlockSpec((1, ho + 3, wo, 48), lambda i: (i, 0, 0, 0)),
            pl.BlockSpec(w4.shape, lambda i: (0, 0, 0)),
            pl.BlockSpec(bias.shape, lambda i: (0, 0)),
        ],
        out_specs=pl.BlockSpec((1, hp + 2, wp + 2, cout), lambda i: (i, 0, 0, 0)),
        out_shape=jax.ShapeDtypeStruct((n, hp + 2, wp + 2, cout), _BF),
        compiler_params=pltpu.CompilerParams(
            dimension_semantics=("parallel",)),
    )(q, w4, bias)


def _conv3x3(x, w_ref, stride, ho, wo):
    nb, hp, wp, c = x.shape
    acc = None
    for dj in range(3):
        if stride == 1:
            xd = x[:, :, dj:dj + wo, :]
        else:
            xd = jnp.stack([
                x[i, :, dj:dj + 2 * wo].reshape(hp, wo, 2, c)[:, :, 0]
                for i in range(nb)])
        for di in range(3):
            if stride == 1:
                lhs = xd[:, di:di + ho]
            else:
                lhs = jnp.stack([
                    xd[i, di:di + 2 * ho].reshape(ho, 2, wo, c)[:, 0]
                    for i in range(nb)])
            d = jnp.dot(lhs.reshape(nb * ho * wo, c), w_ref[3 * di + dj],
                        preferred_element_type=jnp.float32)
            acc = d if acc is None else acc + d
    return acc


def _ds_input(x, ho, wo):
    nb, hp, wp, c = x.shape
    imgs = []
    for i in range(nb):
        cols = x[i, :, 1:1 + 2 * wo].reshape(hp, wo, 2, c)[:, :, 0]
        imgs.append(cols[1:1 + 2 * ho].reshape(ho, 2, wo, c)[:, 0])
    return jnp.stack(imgs).reshape(nb * ho * wo, c)


def _res_block(x, w1, b1, w2, b2, wds, bds, stride, nb):
    n, hp2, wp2, cin = x.shape
    h, w = hp2 - 2, wp2 - 2
    ho, wo = h // stride, w // stride
    cout = w1.shape[2]
    has_ds = wds is not None
    m = nb * ho * wo

    def body(*refs):
        if has_ds:
            x_ref, w1_ref, b1_ref, w2_ref, b2_ref, wds_ref, bds_ref, o_ref = refs
        else:
            x_ref, w1_ref, b1_ref, w2_ref, b2_ref, o_ref = refs
        xv = x_ref[...]
        acc = _conv3x3(xv, w1_ref, stride, ho, wo)
        y1 = jnp.maximum(acc + b1_ref[...], 0.0).astype(_BF)
        y1p = jnp.pad(y1.reshape(nb, ho, wo, cout),
                      ((0, 0), (1, 1), (1, 1), (0, 0)))
        acc2 = _conv3x3(y1p, w2_ref, 1, ho, wo) + b2_ref[...]
        if has_ds:
            dsin = _ds_input(xv, ho, wo)
            idn = jnp.dot(dsin, wds_ref[...],
                          preferred_element_type=jnp.float32) + bds_ref[...]
            idn = idn.astype(_BF)
        else:
            idn = xv[:, 1:1 + h, 1:1 + w, :].reshape(m, cout)
        y2 = jnp.maximum(acc2 + idn.astype(jnp.float32), 0.0).astype(_BF)
        o_ref[...] = jnp.pad(y2.reshape(nb, ho, wo, cout),
                             ((0, 0), (1, 1), (1, 1), (0, 0)))

    in_specs = [
        pl.BlockSpec((nb, hp2, wp2, cin), lambda i: (i, 0, 0, 0)),
        pl.BlockSpec(w1.shape, lambda i: (0, 0, 0)),
        pl.BlockSpec(b1.shape, lambda i: (0, 0)),
        pl.BlockSpec(w2.shape, lambda i: (0, 0, 0)),
        pl.BlockSpec(b2.shape, lambda i: (0, 0)),
    ]
    args = [x, w1, b1, w2, b2]
    if has_ds:
        in_specs += [pl.BlockSpec(wds.shape, lambda i: (0, 0)),
                     pl.BlockSpec(bds.shape, lambda i: (0, 0))]
        args += [wds, bds]

    return pl.pallas_call(
        body,
        grid=(n // nb,),
        in_specs=in_specs,
        out_specs=pl.BlockSpec((nb, ho + 2, wo + 2, cout), lambda i: (i, 0, 0, 0)),
        out_shape=jax.ShapeDtypeStruct((n, ho + 2, wo + 2, cout), _BF),
        compiler_params=pltpu.CompilerParams(
            dimension_semantics=("parallel",)),
    )(*args)


def _head(x, wp, bp, mask):
    n = x.shape[0]
    hw = x.shape[1] - 2
    c = x.shape[3]
    nb = 16 if n % 16 == 0 else n

    def body(x_ref, w_ref, b_ref, m_ref, o_ref):
        xin = x_ref[...][:, 1:1 + hw, 1:1 + hw, :].astype(jnp.float32)
        feat = jnp.mean(xin.reshape(nb, hw * hw, c), axis=1)
        logits = jnp.dot(feat.astype(_BF), w_ref[...],
                         preferred_element_type=jnp.float32) + b_ref[...]
        o_ref[...] = logits * m_ref[...]

    return pl.pallas_call(
        body,
        grid=(n // nb,),
        in_specs=[
            pl.BlockSpec((nb, hw + 2, hw + 2, c), lambda i: (i, 0, 0, 0)),
            pl.BlockSpec(wp.shape, lambda i: (0, 0)),
            pl.BlockSpec(bp.shape, lambda i: (0, 0)),
            pl.BlockSpec((nb, 128), lambda i: (i, 0)),
        ],
        out_specs=pl.BlockSpec((nb, 128), lambda i: (i, 0)),
        out_shape=jax.ShapeDtypeStruct((n, 128), jnp.float32),
        compiler_params=pltpu.CompilerParams(
            dimension_semantics=("parallel",)),
    )(x, wp, bp, mask)


def _forward(p, x_nchw, seed):
    n = x_nchw.shape[0]
    x = jnp.transpose(x_nchw, (0, 2, 3, 1)).astype(_BF)
    xp = jnp.pad(x, ((0, 0), (3, 3), (3, 3), (0, 0)))
    h = x.shape[1]
    ho = (h - 1) // 2 + 1
    hf = (h + 6) // 2

    ph = xp.reshape(n, hf, 2, hf, 2, 3).transpose(0, 1, 3, 2, 4, 5)
    ph = ph.reshape(n, hf, hf, 12)
    q = jnp.concatenate([ph[:, :, d:d + ho, :] for d in range(4)], axis=-1)

    wm1, bb1 = p["conv1"]
    x1 = _conv1_pool(q, wm1, bb1, n, ho, ho)

    def blk(xcur, name, stride, nb):
        prm = p[name]
        wds, bds = prm.get("ds", (None, None))
        return _res_block(xcur, prm["w1"], prm["b1"], prm["w2"], prm["b2"],
                          wds, bds, stride, nb)

    def pick(nb):
        return nb if n % nb == 0 else 1

    x1 = blk(x1, "l1_0", 1, pick(4))
    x1 = blk(x1, "l1_1", 1, pick(4))
    x1 = blk(x1, "l2_0", 2, pick(1))
    x1 = blk(x1, "l2_1", 1, pick(4))
    x1 = blk(x1, "l3_0", 2, pick(4))
    x1 = blk(x1, "l3_1", 1, pick(4))
    x1 = blk(x1, "l4_0", 2, pick(4))
    x1 = blk(x1, "l4_1", 1, pick(4))

    key = jax.random.PRNGKey(seed)
    keep = jax.random.uniform(key, (n, 128)) >= _DR
    mask = keep.astype(jnp.float32) * (1.0 / (1.0 - _DR))

    out = _head(x1, p["fc_wp"], p["fc_bp"], mask)
    return out[:, :_NCLS]


@jax.jit
def _full(conv1_w, conv1_scale, conv1_bias, layer1_0_c1_w, layer1_0_c1_scale, layer1_0_c1_bias, layer1_0_c2_w, layer1_0_c2_scale, layer1_0_c2_bias, layer1_1_c1_w, layer1_1_c1_scale, layer1_1_c1_bias, layer1_1_c2_w, layer1_1_c2_scale, layer1_1_c2_bias, layer2_0_c1_w, layer2_0_c1_scale, layer2_0_c1_bias, layer2_0_c2_w, layer2_0_c2_scale, layer2_0_c2_bias, layer2_0_ds_w, layer2_0_ds_scale, layer2_0_ds_bias, layer2_1_c1_w, layer2_1_c1_scale, layer2_1_c1_bias, layer2_1_c2_w, layer2_1_c2_scale, layer2_1_c2_bias, layer3_0_c1_w, layer3_0_c1_scale, layer3_0_c1_bias, layer3_0_c2_w, layer3_0_c2_scale, layer3_0_c2_bias, layer3_0_ds_w, layer3_0_ds_scale, layer3_0_ds_bias, layer3_1_c1_w, layer3_1_c1_scale, layer3_1_c1_bias, layer3_1_c2_w, layer3_1_c2_scale, layer3_1_c2_bias, layer4_0_c1_w, layer4_0_c1_scale, layer4_0_c1_bias, layer4_0_c2_w, layer4_0_c2_scale, layer4_0_c2_bias, layer4_0_ds_w, layer4_0_ds_scale, layer4_0_ds_bias, layer4_1_c1_w, layer4_1_c1_scale, layer4_1_c1_bias, layer4_1_c2_w, layer4_1_c2_scale, layer4_1_c2_bias, fc_w, fc_b, x_nchw, seed):
    wt = jnp.transpose(conv1_w, (2, 3, 1, 0)) * conv1_scale
    wt = jnp.pad(wt, ((0, 1), (0, 1), (0, 0), (0, 0)))
    wt = wt.reshape(4, 2, 4, 2, 3, conv1_w.shape[0]).transpose(0, 2, 1, 3, 4, 5)
    p = {"conv1": (wt.reshape(4, 48, conv1_w.shape[0]).astype(_BF), conv1_bias)}

    def cb(pref, w1, s1, b1, w2, s2, b2, wds=None, sds=None, bds=None):
        d = {"w1": _fold_w3(w1, s1), "b1": b1,
             "w2": _fold_w3(w2, s2), "b2": b2}
        if wds is not None:
            d["ds"] = (_fold_w(wds, sds), bds)
        return d

    p["l1_0"] = cb("l1_0", layer1_0_c1_w, layer1_0_c1_scale, layer1_0_c1_bias,
                   layer1_0_c2_w, layer1_0_c2_scale, layer1_0_c2_bias)
    p["l1_1"] = cb("l1_1", layer1_1_c1_w, layer1_1_c1_scale, layer1_1_c1_bias,
                   layer1_1_c2_w, layer1_1_c2_scale, layer1_1_c2_bias)
    p["l2_0"] = cb("l2_0", layer2_0_c1_w, layer2_0_c1_scale, layer2_0_c1_bias,
                   layer2_0_c2_w, layer2_0_c2_scale, layer2_0_c2_bias,
                   layer2_0_ds_w, layer2_0_ds_scale, layer2_0_ds_bias)
    p["l2_1"] = cb("l2_1", layer2_1_c1_w, layer2_1_c1_scale, layer2_1_c1_bias,
                   layer2_1_c2_w, layer2_1_c2_scale, layer2_1_c2_bias)
    p["l3_0"] = cb("l3_0", layer3_0_c1_w, layer3_0_c1_scale, layer3_0_c1_bias,
                   layer3_0_c2_w, layer3_0_c2_scale, layer3_0_c2_bias,
                   layer3_0_ds_w, layer3_0_ds_scale, layer3_0_ds_bias)
    p["l3_1"] = cb("l3_1", layer3_1_c1_w, layer3_1_c1_scale, layer3_1_c1_bias,
                   layer3_1_c2_w, layer3_1_c2_scale, layer3_1_c2_bias)
    p["l4_0"] = cb("l4_0", layer4_0_c1_w, layer4_0_c1_scale, layer4_0_c1_bias,
                   layer4_0_c2_w, layer4_0_c2_scale, layer4_0_c2_bias,
                   layer4_0_ds_w, layer4_0_ds_scale, layer4_0_ds_bias)
    p["l4_1"] = cb("l4_1", layer4_1_c1_w, layer4_1_c1_scale, layer4_1_c1_bias,
                   layer4_1_c2_w, layer4_1_c2_scale, layer4_1_c2_bias)

    c = fc_w.shape[0]
    p["fc_wp"] = jnp.zeros((c, 128), _BF).at[:, :_NCLS].set(fc_w.astype(_BF))
    p["fc_bp"] = jnp.zeros((1, 128), jnp.float32).at[:, :_NCLS].set(
        fc_b.reshape(1, -1).astype(jnp.float32))

    return _forward(p, x_nchw, seed)


def kernel(*args, **kwargs):
    return _full(*args, **kwargs)


# speedup vs baseline: 1.0089x; 1.0089x over previous
import jax
import jax.numpy as jnp
from jax.experimental import pallas as pl
from jax.experimental.pallas import tpu as pltpu

_BF = jnp.bfloat16
_NCLS = 10
_DR = 0.2


def _fold_w(w, scale):
    kh, kw = w.shape[2], w.shape[3]
    wt = jnp.transpose(w, (2, 3, 1, 0)).reshape(kh * kw * w.shape[1], w.shape[0])
    return (wt * scale).astype(_BF)


def _fold_w3(w, scale):
    wt = jnp.transpose(w, (2, 3, 1, 0)).reshape(9, w.shape[1], w.shape[0])
    return (wt * scale).astype(_BF)


def _conv1_pool(q, w4, bias, n, ho, wo):
    hp, wp = ho // 2, wo // 2
    cout = w4.shape[2]

    def body(q_ref, w_ref, b_ref, o_ref):
        qv = q_ref[...][0]
        acc = None
        for d in range(4):
            lhs = qv[d:d + ho].reshape(ho * wo, 48)
            z = jnp.dot(lhs, w_ref[d], preferred_element_type=jnp.float32)
            acc = z if acc is None else acc + z
        y = jnp.maximum(acc + b_ref[...], 0.0).astype(_BF).reshape(ho, wo, cout)
        neg = jnp.asarray(-jnp.inf, _BF)
        yp = jnp.pad(y, ((1, 1), (1, 1), (0, 0)), constant_values=neg)
        cmax = None
        for j in range(3):
            cj = yp[:, j:j + 2 * wp].reshape(ho + 2, wp, 2, cout)[:, :, 0]
            cmax = cj if cmax is None else jnp.maximum(cmax, cj)
        out = None
        for i in range(3):
            ri = cmax[i:i + 2 * hp].reshape(hp, 2, wp, cout)[:, 0]
            out = ri if out is None else jnp.maximum(out, ri)
        o_ref[...] = jnp.pad(out, ((1, 1), (1, 1), (0, 0)))[None]

    return pl.pallas_call(
        body,
        grid=(n,),
        in_specs=[
            pl.BlockSpec((1, ho + 3, wo, 48), lambda i: (i, 0, 0, 0)),
            pl.BlockSpec(w4.shape, lambda i: (0, 0, 0)),
            pl.BlockSpec(bias.shape, lambda i: (0, 0)),
        ],
        out_specs=pl.BlockSpec((1, hp + 2, wp + 2, cout), lambda i: (i, 0, 0, 0)),
        out_shape=jax.ShapeDtypeStruct((n, hp + 2, wp + 2, cout), _BF),
        compiler_params=pltpu.CompilerParams(
            dimension_semantics=("parallel",)),
    )(q, w4, bias)


def _conv3x3(x, w_ref, stride, ho, wo):
    nb, hp, wp, c = x.shape
    acc = None
    for dj in range(3):
        if stride == 1:
            xd = x[:, :, dj:dj + wo, :]
        else:
            xd = jnp.stack([
                x[i, :, dj:dj + 2 * wo].reshape(hp, wo, 2, c)[:, :, 0]
                for i in range(nb)])
        for di in range(3):
            if stride == 1:
                lhs = xd[:, di:di + ho]
            else:
                lhs = jnp.stack([
                    xd[i, di:di + 2 * ho].reshape(ho, 2, wo, c)[:, 0]
                    for i in range(nb)])
            d = jnp.dot(lhs.reshape(nb * ho * wo, c), w_ref[3 * di + dj],
                        preferred_element_type=jnp.float32)
            acc = d if acc is None else acc + d
    return acc


def _ds_input(x, ho, wo):
    nb, hp, wp, c = x.shape
    imgs = []
    for i in range(nb):
        cols = x[i, :, 1:1 + 2 * wo].reshape(hp, wo, 2, c)[:, :, 0]
        imgs.append(cols[1:1 + 2 * ho].reshape(ho, 2, wo, c)[:, 0])
    return jnp.stack(imgs).reshape(nb * ho * wo, c)


def _res_block(x, w1, b1, w2, b2, wds, bds, stride, nb):
    n, hp2, wp2, cin = x.shape
    h, w = hp2 - 2, wp2 - 2
    ho, wo = h // stride, w // stride
    cout = w1.shape[2]
    has_ds = wds is not None
    m = nb * ho * wo

    def body(*refs):
        if has_ds:
            x_ref, w1_ref, b1_ref, w2_ref, b2_ref, wds_ref, bds_ref, o_ref = refs
        else:
            x_ref, w1_ref, b1_ref, w2_ref, b2_ref, o_ref = refs
        xv = x_ref[...]
        acc = _conv3x3(xv, w1_ref, stride, ho, wo)
        y1 = jnp.maximum(acc + b1_ref[...], 0.0).astype(_BF)
        y1p = jnp.pad(y1.reshape(nb, ho, wo, cout),
                      ((0, 0), (1, 1), (1, 1), (0, 0)))
        acc2 = _conv3x3(y1p, w2_ref, 1, ho, wo) + b2_ref[...]
        if has_ds:
            dsin = _ds_input(xv, ho, wo)
            idn = jnp.dot(dsin, wds_ref[...],
                          preferred_element_type=jnp.float32) + bds_ref[...]
            idn = idn.astype(_BF)
        else:
            idn = xv[:, 1:1 + h, 1:1 + w, :].reshape(m, cout)
        y2 = jnp.maximum(acc2 + idn.astype(jnp.float32), 0.0).astype(_BF)
        o_ref[...] = jnp.pad(y2.reshape(nb, ho, wo, cout),
                             ((0, 0), (1, 1), (1, 1), (0, 0)))

    in_specs = [
        pl.BlockSpec((nb, hp2, wp2, cin), lambda i: (i, 0, 0, 0)),
        pl.BlockSpec(w1.shape, lambda i: (0, 0, 0)),
        pl.BlockSpec(b1.shape, lambda i: (0, 0)),
        pl.BlockSpec(w2.shape, lambda i: (0, 0, 0)),
        pl.BlockSpec(b2.shape, lambda i: (0, 0)),
    ]
    args = [x, w1, b1, w2, b2]
    if has_ds:
        in_specs += [pl.BlockSpec(wds.shape, lambda i: (0, 0)),
                     pl.BlockSpec(bds.shape, lambda i: (0, 0))]
        args += [wds, bds]

    return pl.pallas_call(
        body,
        grid=(n // nb,),
        in_specs=in_specs,
        out_specs=pl.BlockSpec((nb, ho + 2, wo + 2, cout), lambda i: (i, 0, 0, 0)),
        out_shape=jax.ShapeDtypeStruct((n, ho + 2, wo + 2, cout), _BF),
        compiler_params=pltpu.CompilerParams(
            dimension_semantics=("parallel",)),
    )(*args)


def _head(x, wp, bp, mask):
    n = x.shape[0]
    hw = x.shape[1] - 2
    c = x.shape[3]
    nb = 16 if n % 16 == 0 else n

    def body(x_ref, w_ref, b_ref, m_ref, o_ref):
        xin = x_ref[...][:, 1:1 + hw, 1:1 + hw, :].astype(jnp.float32)
        feat = jnp.mean(xin.reshape(nb, hw * hw, c), axis=1)
        logits = jnp.dot(feat.astype(_BF), w_ref[...],
                         preferred_element_type=jnp.float32) + b_ref[...]
        o_ref[...] = logits * m_ref[...]

    return pl.pallas_call(
        body,
        grid=(n // nb,),
        in_specs=[
            pl.BlockSpec((nb, hw + 2, hw + 2, c), lambda i: (i, 0, 0, 0)),
            pl.BlockSpec(wp.shape, lambda i: (0, 0)),
            pl.BlockSpec(bp.shape, lambda i: (0, 0)),
            pl.BlockSpec((nb, 128), lambda i: (i, 0)),
        ],
        out_specs=pl.BlockSpec((nb, 128), lambda i: (i, 0)),
        out_shape=jax.ShapeDtypeStruct((n, 128), jnp.float32),
        compiler_params=pltpu.CompilerParams(
            dimension_semantics=("parallel",)),
    )(x, wp, bp, mask)


def _forward(p, x_nchw, seed):
    n = x_nchw.shape[0]
    x = jnp.transpose(x_nchw, (0, 2, 3, 1)).astype(_BF)
    xp = jnp.pad(x, ((0, 0), (3, 3), (3, 3), (0, 0)))
    h = x.shape[1]
    ho = (h - 1) // 2 + 1
    hf = (h + 6) // 2

    ph = xp.reshape(n, hf, 2, hf, 2, 3).transpose(0, 1, 3, 2, 4, 5)
    ph = ph.reshape(n, hf, hf, 12)
    q = jnp.concatenate([ph[:, :, d:d + ho, :] for d in range(4)], axis=-1)

    wm1, bb1 = p["conv1"]
    x1 = _conv1_pool(q, wm1, bb1, n, ho, ho)

    def blk(xcur, name, stride, nb):
        prm = p[name]
        wds, bds = prm.get("ds", (None, None))
        return _res_block(xcur, prm["w1"], prm["b1"], prm["w2"], prm["b2"],
                          wds, bds, stride, nb)

    def pick(nb):
        return nb if n % nb == 0 else 1

    x1 = blk(x1, "l1_0", 1, pick(1))
    x1 = blk(x1, "l1_1", 1, pick(1))
    x1 = blk(x1, "l2_0", 2, pick(1))
    x1 = blk(x1, "l2_1", 1, pick(1))
    x1 = blk(x1, "l3_0", 2, pick(4))
    x1 = blk(x1, "l3_1", 1, pick(4))
    x1 = blk(x1, "l4_0", 2, pick(4))
    x1 = blk(x1, "l4_1", 1, pick(4))

    key = jax.random.PRNGKey(seed)
    keep = jax.random.uniform(key, (n, 128)) >= _DR
    mask = keep.astype(jnp.float32) * (1.0 / (1.0 - _DR))

    out = _head(x1, p["fc_wp"], p["fc_bp"], mask)
    return out[:, :_NCLS]


@jax.jit
def _full(conv1_w, conv1_scale, conv1_bias, layer1_0_c1_w, layer1_0_c1_scale, layer1_0_c1_bias, layer1_0_c2_w, layer1_0_c2_scale, layer1_0_c2_bias, layer1_1_c1_w, layer1_1_c1_scale, layer1_1_c1_bias, layer1_1_c2_w, layer1_1_c2_scale, layer1_1_c2_bias, layer2_0_c1_w, layer2_0_c1_scale, layer2_0_c1_bias, layer2_0_c2_w, layer2_0_c2_scale, layer2_0_c2_bias, layer2_0_ds_w, layer2_0_ds_scale, layer2_0_ds_bias, layer2_1_c1_w, layer2_1_c1_scale, layer2_1_c1_bias, layer2_1_c2_w, layer2_1_c2_scale, layer2_1_c2_bias, layer3_0_c1_w, layer3_0_c1_scale, layer3_0_c1_bias, layer3_0_c2_w, layer3_0_c2_scale, layer3_0_c2_bias, layer3_0_ds_w, layer3_0_ds_scale, layer3_0_ds_bias, layer3_1_c1_w, layer3_1_c1_scale, layer3_1_c1_bias, layer3_1_c2_w, layer3_1_c2_scale, layer3_1_c2_bias, layer4_0_c1_w, layer4_0_c1_scale, layer4_0_c1_bias, layer4_0_c2_w, layer4_0_c2_scale, layer4_0_c2_bias, layer4_0_ds_w, layer4_0_ds_scale, layer4_0_ds_bias, layer4_1_c1_w, layer4_1_c1_scale, layer4_1_c1_bias, layer4_1_c2_w, layer4_1_c2_scale, layer4_1_c2_bias, fc_w, fc_b, x_nchw, seed):
    wt = jnp.transpose(conv1_w, (2, 3, 1, 0)) * conv1_scale
    wt = jnp.pad(wt, ((0, 1), (0, 1), (0, 0), (0, 0)))
    wt = wt.reshape(4, 2, 4, 2, 3, conv1_w.shape[0]).transpose(0, 2, 1, 3, 4, 5)
    p = {"conv1": (wt.reshape(4, 48, conv1_w.shape[0]).astype(_BF), conv1_bias)}

    def cb(pref, w1, s1, b1, w2, s2, b2, wds=None, sds=None, bds=None):
        d = {"w1": _fold_w3(w1, s1), "b1": b1,
             "w2": _fold_w3(w2, s2), "b2": b2}
        if wds is not None:
            d["ds"] = (_fold_w(wds, sds), bds)
        return d

    p["l1_0"] = cb("l1_0", layer1_0_c1_w, layer1_0_c1_scale, layer1_0_c1_bias,
                   layer1_0_c2_w, layer1_0_c2_scale, layer1_0_c2_bias)
    p["l1_1"] = cb("l1_1", layer1_1_c1_w, layer1_1_c1_scale, layer1_1_c1_bias,
                   layer1_1_c2_w, layer1_1_c2_scale, layer1_1_c2_bias)
    p["l2_0"] = cb("l2_0", layer2_0_c1_w, layer2_0_c1_scale, layer2_0_c1_bias,
                   layer2_0_c2_w, layer2_0_c2_scale, layer2_0_c2_bias,
                   layer2_0_ds_w, layer2_0_ds_scale, layer2_0_ds_bias)
    p["l2_1"] = cb("l2_1", layer2_1_c1_w, layer2_1_c1_scale, layer2_1_c1_bias,
                   layer2_1_c2_w, layer2_1_c2_scale, layer2_1_c2_bias)
    p["l3_0"] = cb("l3_0", layer3_0_c1_w, layer3_0_c1_scale, layer3_0_c1_bias,
                   layer3_0_c2_w, layer3_0_c2_scale, layer3_0_c2_bias,
                   layer3_0_ds_w, layer3_0_ds_scale, layer3_0_ds_bias)
    p["l3_1"] = cb("l3_1", layer3_1_c1_w, layer3_1_c1_scale, layer3_1_c1_bias,
                   layer3_1_c2_w, layer3_1_c2_scale, layer3_1_c2_bias)
    p["l4_0"] = cb("l4_0", layer4_0_c1_w, layer4_0_c1_scale, layer4_0_c1_bias,
                   layer4_0_c2_w, layer4_0_c2_scale, layer4_0_c2_bias,
                   layer4_0_ds_w, layer4_0_ds_scale, layer4_0_ds_bias)
    p["l4_1"] = cb("l4_1", layer4_1_c1_w, layer4_1_c1_scale, layer4_1_c1_bias,
                   layer4_1_c2_w, layer4_1_c2_scale, layer4_1_c2_bias)

    c = fc_w.shape[0]
    p["fc_wp"] = jnp.zeros((c, 128), _BF).at[:, :_NCLS].set(fc_w.astype(_BF))
    p["fc_bp"] = jnp.zeros((1, 128), jnp.float32).at[:, :_NCLS].set(
        fc_b.reshape(1, -1).astype(jnp.float32))

    return _forward(p, x_nchw, seed)


def kernel(*args, **kwargs):
    return _full(*args, **kwargs)
